# Initial kernel scaffold; baseline (speedup 1.0000x reference)
import jax
import jax.numpy as jnp
from jax.experimental import pallas as pl


def kernel(ke, ke_invidx, step):
    raise NotImplementedError("write your pallas kernel here")


# speedup vs baseline: 113.1357x; 113.1357x over previous
import functools

import jax
import jax.numpy as jnp
from jax import lax
from jax.experimental import pallas as pl
from jax.experimental.pallas import tpu as pltpu
from jax.experimental.pallas import tpu_sc as plsc

NBASIS = 1000000
NUNIQ = 65536

_NC = 2
_NS = 16
_NW = _NC * _NS
_LANES = 16

_PER_TILE = 31264
_NP = _NW * _PER_TILE

_mesh = plsc.VectorSubcoreMesh(core_axis_name="c", subcore_axis_name="s")


@functools.partial(
    pl.kernel,
    mesh=_mesh,
    out_type=jax.ShapeDtypeStruct((_NP,), jnp.float32),
    scratch_types=[
        pltpu.VMEM((NUNIQ,), jnp.float32),
        pltpu.VMEM((_PER_TILE,), jnp.int32),
        pltpu.VMEM((_PER_TILE,), jnp.float32),
        pltpu.VMEM((_LANES,), jnp.float32),
    ],
    compiler_params=pltpu.CompilerParams(needs_layout_passes=False),
)
def _sc_gather(ke_hbm, idx_hbm, step_hbm, out_hbm, tab_v, idx_v, out_v, step_v):
    wid = lax.axis_index("s") * _NC + lax.axis_index("c")
    base = wid * _PER_TILE
    pltpu.sync_copy(step_hbm, step_v)
    pltpu.sync_copy(ke_hbm, tab_v)
    pltpu.sync_copy(idx_hbm.at[pl.ds(base, _PER_TILE)], idx_v)
    sv = step_v[...]

    def body(i, _):
        off = pl.multiple_of(i * _LANES, _LANES)
        iv = idx_v[pl.ds(off, _LANES)]
        vals = plsc.load_gather(tab_v, [iv])
        out_v[pl.ds(off, _LANES)] = vals * sv
        return 0

    lax.fori_loop(0, _PER_TILE // _LANES, body, 0)
    pltpu.sync_copy(out_v, out_hbm.at[pl.ds(base, _PER_TILE)])


def kernel(ke, ke_invidx, step):
    idx = jnp.pad(ke_invidx.astype(jnp.int32), (0, _NP - NBASIS))
    step_vec = jnp.full((_LANES,), step, dtype=jnp.float32)
    out = _sc_gather(ke, idx, step_vec)
    return out[:NBASIS]
